# Initial kernel scaffold; baseline (speedup 1.0000x reference)
import jax
import jax.numpy as jnp
from jax.experimental import pallas as pl


def kernel(features, edge_index, Ws0, Wn0, b0, g0, be0, Ws1, Wn1, b1, g1, be1, Ws2, Wn2, b2, g2, be2, Wc, bc):
    raise NotImplementedError("write your pallas kernel here")


# speedup vs baseline: 6.9579x; 6.9579x over previous
import functools

import jax
import jax.numpy as jnp
from jax import lax
from jax.experimental import pallas as pl
from jax.experimental.pallas import tpu as pltpu
from jax.experimental.pallas import tpu_sc as plsc

N = 10000
E = 320000
H = 128

NC = 2
NS = 16
NW = NC * NS
E_PER_W = E // NW
CHUNK = 80
N_CHUNKS = E_PER_W // CHUNK
N_PAD = 10240
ROWS_PER_TILE = N_PAD // NS
DEG_W = 128


def _sc_agg_body(h_hbm, ei_hbm, zrows_hbm, aggp_hbm, src_v, dst_v, rows_v,
                 acc_sh):
    c = lax.axis_index("c")
    s = lax.axis_index("s")
    w = c * NS + s

    pltpu.sync_copy(zrows_hbm, acc_sh.at[pl.ds(s * ROWS_PER_TILE, ROWS_PER_TILE)])
    pltpu.sync_copy(ei_hbm.at[0, w], src_v)
    pltpu.sync_copy(ei_hbm.at[1, w], dst_v)
    plsc.subcore_barrier()

    @pl.loop(0, N_CHUNKS)
    def _(j):
        pltpu.sync_copy(h_hbm.at[src_v.at[j]], rows_v)
        pltpu.sync_copy(rows_v, acc_sh.at[dst_v.at[j]], add=True)

    plsc.subcore_barrier()

    sl = pl.ds(s * ROWS_PER_TILE, ROWS_PER_TILE)
    pltpu.sync_copy(acc_sh.at[sl], aggp_hbm.at[c, sl])


@functools.lru_cache(maxsize=None)
def _get_sc_agg():
    return pl.kernel(
        _sc_agg_body,
        out_type=jax.ShapeDtypeStruct((NC, N_PAD, H), jnp.float32),
        mesh=plsc.VectorSubcoreMesh(core_axis_name="c", subcore_axis_name="s"),
        scratch_types=[
            pltpu.VMEM((N_CHUNKS, CHUNK), jnp.int32),
            pltpu.VMEM((N_CHUNKS, CHUNK), jnp.int32),
            pltpu.VMEM((CHUNK, H), jnp.float32),
            pltpu.VMEM_SHARED((N_PAD, H), jnp.float32),
        ],
        name="sc_agg",
    )


def _sc_deg_body(ei_hbm, zdeg_hbm, ones_hbm, degp_hbm, dst_v, ones_v, deg_sh):
    c = lax.axis_index("c")
    s = lax.axis_index("s")
    w = c * NS + s

    pltpu.sync_copy(zdeg_hbm, deg_sh.at[pl.ds(s * ROWS_PER_TILE, ROWS_PER_TILE)])
    pltpu.sync_copy(ei_hbm.at[1, w], dst_v)
    pltpu.sync_copy(ones_hbm, ones_v)
    plsc.subcore_barrier()

    @pl.loop(0, N_CHUNKS)
    def _(j):
        pltpu.sync_copy(ones_v, deg_sh.at[dst_v.at[j]], add=True)

    plsc.subcore_barrier()
    sl = pl.ds(s * ROWS_PER_TILE, ROWS_PER_TILE)
    pltpu.sync_copy(deg_sh.at[sl], degp_hbm.at[c, sl])


@functools.lru_cache(maxsize=None)
def _get_sc_deg():
    return pl.kernel(
        _sc_deg_body,
        out_type=jax.ShapeDtypeStruct((NC, N_PAD, DEG_W), jnp.float32),
        mesh=plsc.VectorSubcoreMesh(core_axis_name="c", subcore_axis_name="s"),
        scratch_types=[
            pltpu.VMEM((N_CHUNKS, CHUNK), jnp.int32),
            pltpu.VMEM((CHUNK, DEG_W), jnp.float32),
            pltpu.VMEM_SHARED((N_PAD, DEG_W), jnp.float32),
        ],
        name="sc_deg",
    )


def _tc_layer_body(final, h_ref, a_ref, d_ref, ws_ref, wn_ref, b_ref,
                   g_ref, be_ref, wc_ref, bc_ref, o_ref):
    agg = a_ref[0, 0:N, :] + a_ref[1, 0:N, :]
    deg = d_ref[0, 0:N, 0:1] + d_ref[1, 0:N, 0:1]
    mean = agg / jnp.maximum(deg, 1.0)
    h = h_ref[...]
    z = (jnp.dot(h, ws_ref[...], preferred_element_type=jnp.float32)
         + jnp.dot(mean, wn_ref[...], preferred_element_type=jnp.float32)
         + b_ref[...])
    m = jnp.mean(z, axis=0, keepdims=True)
    v = jnp.mean((z - m) * (z - m), axis=0, keepdims=True)
    zn = (z - m) * lax.rsqrt(v + 1e-5)
    act = jnp.maximum(g_ref[...] * zn + be_ref[...], 0.0)
    if final:
        o_ref[...] = (jnp.dot(act, wc_ref[...], preferred_element_type=jnp.float32)
                      + bc_ref[...])
    else:
        o_ref[...] = act


def _tc_layer(h, aggp, degp, Ws, Wn, b, g, be, wc_pad, bc_pad, final):
    return pl.pallas_call(
        functools.partial(_tc_layer_body, final),
        out_shape=jax.ShapeDtypeStruct((N, H), jnp.float32),
    )(h, aggp, degp, Ws, Wn, b.reshape(1, H), g.reshape(1, H),
      be.reshape(1, H), wc_pad, bc_pad)


def kernel(features, edge_index, Ws0, Wn0, b0, g0, be0, Ws1, Wn1, b1, g1,
           be1, Ws2, Wn2, b2, g2, be2, Wc, bc):
    ei = edge_index.reshape(2, NW, N_CHUNKS, CHUNK)
    zrows = jnp.zeros((ROWS_PER_TILE, H), jnp.float32)
    ones = jnp.ones((CHUNK, DEG_W), jnp.float32)
    wc_pad = jnp.zeros((H, H), jnp.float32).at[:, :Wc.shape[1]].set(Wc)
    bc_pad = jnp.zeros((1, H), jnp.float32).at[0, :bc.shape[0]].set(bc)

    h = features
    degp = _get_sc_deg()(ei, zrows, ones)
    aggp = _get_sc_agg()(h, ei, zrows)
    h = _tc_layer(h, aggp, degp, Ws0, Wn0, b0, g0, be0, wc_pad, bc_pad, False)
    aggp = _get_sc_agg()(h, ei, zrows)
    h = _tc_layer(h, aggp, degp, Ws1, Wn1, b1, g1, be1, wc_pad, bc_pad, False)
    aggp = _get_sc_agg()(h, ei, zrows)
    out = _tc_layer(h, aggp, degp, Ws2, Wn2, b2, g2, be2, wc_pad, bc_pad, True)
    return out[:, :Wc.shape[1]]


# speedup vs baseline: 10.9951x; 1.5802x over previous
import functools

import jax
import jax.numpy as jnp
from jax import lax
from jax.experimental import pallas as pl
from jax.experimental.pallas import tpu as pltpu
from jax.experimental.pallas import tpu_sc as plsc

N = 10000
E = 320000
H = 128

NC = 2
NS = 16
NW = NC * NS
E_PER_W = E // NW
CHUNK = 40
N_CHUNKS = E_PER_W // CHUNK
NBUF = 5
BLK_CH = 50
NBLKS = N_CHUNKS // BLK_CH
N_PAD = 10240
ROWS_PER_TILE = N_PAD // NS
DEG_W = 128


def _sc_agg_body(h_hbm, ei_hbm, zrows_hbm, aggp_hbm, srcb, dstb, acc_sh,
                 *ring):
    rows = ring[:NBUF]
    sems = ring[NBUF:]
    c = lax.axis_index("c")
    s = lax.axis_index("s")
    w = c * NS + s

    pltpu.sync_copy(zrows_hbm, acc_sh.at[pl.ds(s * ROWS_PER_TILE, ROWS_PER_TILE)])
    plsc.subcore_barrier()

    @pl.loop(0, NBLKS)
    def _(k):
        pltpu.sync_copy(ei_hbm.at[0, w, k], srcb)
        pltpu.sync_copy(ei_hbm.at[1, w, k], dstb)

        for b in range(NBUF):
            pltpu.async_copy(h_hbm.at[srcb.at[b]], rows[b], sems[b])

        @pl.loop(0, BLK_CH - NBUF, step=NBUF)
        def _(g):
            for b in range(NBUF):
                pltpu.make_async_copy(h_hbm.at[srcb.at[b]], rows[b],
                                      sems[b]).wait()
                pltpu.sync_copy(rows[b], acc_sh.at[dstb.at[g + b]], add=True)
                pltpu.async_copy(h_hbm.at[srcb.at[g + NBUF + b]], rows[b],
                                 sems[b])

        for b in range(NBUF):
            pltpu.make_async_copy(h_hbm.at[srcb.at[b]], rows[b],
                                  sems[b]).wait()
            pltpu.sync_copy(rows[b], acc_sh.at[dstb.at[BLK_CH - NBUF + b]],
                            add=True)

    plsc.subcore_barrier()

    sl = pl.ds(s * ROWS_PER_TILE, ROWS_PER_TILE)
    pltpu.sync_copy(acc_sh.at[sl], aggp_hbm.at[c, sl])


@functools.lru_cache(maxsize=None)
def _get_sc_agg():
    return pl.kernel(
        _sc_agg_body,
        out_type=jax.ShapeDtypeStruct((NC, N_PAD, H), jnp.float32),
        mesh=plsc.VectorSubcoreMesh(core_axis_name="c", subcore_axis_name="s"),
        scratch_types=[
            pltpu.VMEM((BLK_CH, CHUNK), jnp.int32),
            pltpu.VMEM((BLK_CH, CHUNK), jnp.int32),
            pltpu.VMEM_SHARED((N_PAD, H), jnp.float32),
        ] + [pltpu.VMEM((CHUNK, H), jnp.float32)] * NBUF
          + [pltpu.SemaphoreType.DMA] * NBUF,
        name="sc_agg",
    )


def _sc_deg_body(ei_hbm, zdeg_hbm, ones_hbm, degp_hbm, dst_v, ones_v, deg_sh):
    c = lax.axis_index("c")
    s = lax.axis_index("s")
    w = c * NS + s

    pltpu.sync_copy(zdeg_hbm, deg_sh.at[pl.ds(s * ROWS_PER_TILE, ROWS_PER_TILE)])
    pltpu.sync_copy(ei_hbm.at[1, w], dst_v)
    pltpu.sync_copy(ones_hbm, ones_v)
    plsc.subcore_barrier()

    @pl.loop(0, N_CHUNKS)
    def _(j):
        pltpu.sync_copy(ones_v, deg_sh.at[dst_v.at[j]], add=True)

    plsc.subcore_barrier()
    sl = pl.ds(s * ROWS_PER_TILE, ROWS_PER_TILE)
    pltpu.sync_copy(deg_sh.at[sl], degp_hbm.at[c, sl])


@functools.lru_cache(maxsize=None)
def _get_sc_deg():
    return pl.kernel(
        _sc_deg_body,
        out_type=jax.ShapeDtypeStruct((NC, N_PAD, DEG_W), jnp.float32),
        mesh=plsc.VectorSubcoreMesh(core_axis_name="c", subcore_axis_name="s"),
        scratch_types=[
            pltpu.VMEM((N_CHUNKS, CHUNK), jnp.int32),
            pltpu.VMEM((CHUNK, DEG_W), jnp.float32),
            pltpu.VMEM_SHARED((N_PAD, DEG_W), jnp.float32),
        ],
        name="sc_deg",
    )


def _tc_layer_body(final, h_ref, a_ref, d_ref, ws_ref, wn_ref, b_ref,
                   g_ref, be_ref, wc_ref, bc_ref, o_ref):
    agg = a_ref[0, 0:N, :] + a_ref[1, 0:N, :]
    deg = d_ref[0, 0:N, 0:1] + d_ref[1, 0:N, 0:1]
    mean = agg / jnp.maximum(deg, 1.0)
    h = h_ref[...]
    z = (jnp.dot(h, ws_ref[...], preferred_element_type=jnp.float32)
         + jnp.dot(mean, wn_ref[...], preferred_element_type=jnp.float32)
         + b_ref[...])
    m = jnp.mean(z, axis=0, keepdims=True)
    v = jnp.mean((z - m) * (z - m), axis=0, keepdims=True)
    zn = (z - m) * lax.rsqrt(v + 1e-5)
    act = jnp.maximum(g_ref[...] * zn + be_ref[...], 0.0)
    if final:
        o_ref[...] = (jnp.dot(act, wc_ref[...], preferred_element_type=jnp.float32)
                      + bc_ref[...])
    else:
        o_ref[...] = act


def _tc_layer(h, aggp, degp, Ws, Wn, b, g, be, wc_pad, bc_pad, final):
    return pl.pallas_call(
        functools.partial(_tc_layer_body, final),
        out_shape=jax.ShapeDtypeStruct((N, H), jnp.float32),
    )(h, aggp, degp, Ws, Wn, b.reshape(1, H), g.reshape(1, H),
      be.reshape(1, H), wc_pad, bc_pad)


def kernel(features, edge_index, Ws0, Wn0, b0, g0, be0, Ws1, Wn1, b1, g1,
           be1, Ws2, Wn2, b2, g2, be2, Wc, bc):
    ei = edge_index.reshape(2, NW, NBLKS, BLK_CH, CHUNK)
    ei_flat = edge_index.reshape(2, NW, N_CHUNKS, CHUNK)
    zrows = jnp.zeros((ROWS_PER_TILE, H), jnp.float32)
    ones = jnp.ones((CHUNK, DEG_W), jnp.float32)
    wc_pad = jnp.zeros((H, H), jnp.float32).at[:, :Wc.shape[1]].set(Wc)
    bc_pad = jnp.zeros((1, H), jnp.float32).at[0, :bc.shape[0]].set(bc)

    h = features
    degp = _get_sc_deg()(ei_flat, zrows, ones)
    aggp = _get_sc_agg()(h, ei, zrows)
    h = _tc_layer(h, aggp, degp, Ws0, Wn0, b0, g0, be0, wc_pad, bc_pad, False)
    aggp = _get_sc_agg()(h, ei, zrows)
    h = _tc_layer(h, aggp, degp, Ws1, Wn1, b1, g1, be1, wc_pad, bc_pad, False)
    aggp = _get_sc_agg()(h, ei, zrows)
    out = _tc_layer(h, aggp, degp, Ws2, Wn2, b2, g2, be2, wc_pad, bc_pad, True)
    return out[:, :Wc.shape[1]]
